# baseline (speedup 1.0000x reference)
import jax
import jax.numpy as jnp
from jax.experimental import pallas as pl

_LATENT = 10
_K = 5
_GAMMA = 0.9


def _blk(p, x):
    x = x @ p['W1'].T + p['b1']
    x = jax.nn.leaky_relu(x, 0.01)
    x = x @ p['W2'].T + p['b2']
    x = jax.nn.leaky_relu(x, 0.01)
    return x @ p['W4'].T + p['b4']


def kernel(buses, lines, generators, B, L, G, params):
    n = buses.shape[0]
    src = lines[:, L['f_bus']].astype(jnp.int32) - 1
    dst = lines[:, L['t_bus']].astype(jnp.int32) - 1
    gen_idcs = generators[:, G['bus_i']].astype(jnp.int32) - 1
    seg = lambda x, i: jax.ops.segment_sum(x, i, num_segments=n)
    order = jnp.argsort(dst, stable=True)
    sdst = dst[order]
    slattr = lines[:, 2:][order]
    sseg = lambda x, i: jax.ops.segment_sum(x, i, num_segments=n,
                                            indices_are_sorted=True)

    m = jnp.zeros((n, _LATENT), jnp.float32)
    theta = jnp.zeros((n,), jnp.float32)
    total_loss = jnp.float32(0.0)
    v = seg(generators[:, G['vg']], gen_idcs)
    v = jnp.where(v == 0, jnp.ones_like(v), v)
    pg_new = seg(generators[:, G['Pg']], gen_idcs)
    delta_p = pg_new - buses[:, B['Pd']] - buses[:, B['Gs']] * v ** 2
    qg_new = seg(generators[:, G['qg']], gen_idcs)
    delta_q = qg_new - buses[:, B['Qd']] + buses[:, B['Bs']] * v ** 2

    for k in range(_K):
        phi_in = jnp.concatenate([m[sdst], slattr], axis=1)
        pv = sseg(_blk(params['phi_v'][k], phi_in), sdst)
        pt = sseg(_blk(params['phi_theta'][k], phi_in), sdst)
        pm = sseg(_blk(params['phi_m'][k], phi_in), sdst)
        nf = jnp.stack([v, theta, delta_p, delta_q], axis=1)
        theta = theta + _blk(params['L_theta'][k], jnp.concatenate([nf, m, pt], 1))[:, 0]
        v = v + _blk(params['L_v'][k], jnp.concatenate([nf, m, pv], 1))[:, 0]
        m = _blk(params['L_m'][k], jnp.concatenate([nf, m, pm], 1))

        r = lines[:, L['r']]; xr = lines[:, L['x']]; tau = lines[:, L['tau']]
        bsh = lines[:, L['b']]; tsh = lines[:, L['theta']]
        y = 1.0 / jnp.sqrt(r ** 2 + xr ** 2)
        d_ij = theta[src] - theta[dst]
        msg = jnp.abs(v[src] * v[dst] * y[src] / tau[src] * (jnp.sin(theta[src] - theta[dst] - d_ij[src] - tsh[src]) + jnp.sin(theta[dst] - theta[src] - d_ij[src] + tsh[src])) + v[src] / tau[src] ** 2 * y[src] * jnp.sin(d_ij[src]) + v[dst] ** 2 * y[src] * jnp.sin(d_ij[src]))
        p_joule = jnp.sum(seg(msg, dst))
        p_global = jnp.sum(buses[:, B['Pd']]) + jnp.sum(v ** 2 * buses[:, B['Gs']]) + p_joule
        s_set = jnp.sum(generators[:, G['Pg_set']]); s_min = jnp.sum(generators[:, G['Pmin']]); s_max = jnp.sum(generators[:, G['Pmax']])
        lam = jnp.where(p_global < s_set, (p_global - s_min) / (2 * (s_set - s_min)), (p_global - 2 * s_set + s_max) / (2 * (s_max - s_set)))
        pg_k = jnp.where(lam < 0.5, generators[:, G['Pmin']] + 2 * (generators[:, G['Pg_set']] - generators[:, G['Pmin']]) * lam, 2 * generators[:, G['Pg_set']] - generators[:, G['Pmax']] + 2 * (generators[:, G['Pmax']] - generators[:, G['Pg_set']]) * lam)
        qg_start = buses[:, B['Qd']] - buses[:, B['Bs']] * v ** 2
        d_ji = theta[dst] - theta[src]
        msg_from = -v[src] * v[dst] * y[src] / tau[src] * jnp.cos(theta[src] - theta[dst] - d_ij[src] - tsh[src]) + (v[src] / tau[src]) ** 2 * (y[src] * jnp.cos(d_ij[src]) - bsh[src] / 2)
        msg_to = -v[dst] * v[src] * y[dst] / tau[dst] * jnp.cos(theta[dst] - theta[src] - d_ji[dst] - tsh[dst]) + v[dst] ** 2 * (y[dst] * jnp.sin(d_ji[dst]) - bsh[dst] / 2)
        aggr_from = seg(msg_from, dst)
        aggr_to = seg(msg_to, src)
        qg_k = qg_start - aggr_from - aggr_to

        delta_p_gens = seg(pg_k, gen_idcs)
        delta_p_start = delta_p_gens - buses[:, B['Pd']] - buses[:, B['Gs']] * v ** 2
        delta_q_start = qg_k - buses[:, B['Qd']] + buses[:, B['Bs']] * v ** 2
        p_msg_from = v[src] * v[dst] * y[src] / tau[src] * jnp.sin(theta[src] - theta[dst] - d_ij[src] - tsh[src]) + (v[src] / tau[src]) ** 2 * y[src] * jnp.sin(d_ij[src])
        p_msg_to = v[dst] * v[src] * y[dst] / tau[dst] * jnp.sin(theta[dst] - theta[src] - d_ji[dst] - tsh[dst]) + v[dst] ** 2 * y[dst] * jnp.sin(d_ji[dst])
        delta_p = delta_p_start + seg(p_msg_from, dst) + seg(p_msg_to, src)
        q_msg_from = -v[src] * v[dst] * y[src] / tau[src] * jnp.cos(theta[src] - theta[dst] - d_ij[src] - tsh[src]) + (v[src] / tau[src]) ** 2 * (y[src] * jnp.cos(d_ij[src]) - bsh[src] / 2)
        q_msg_to = -v[dst] * v[src] * y[dst] / tau[dst] * jnp.cos(theta[dst] - theta[src] - d_ji[dst] - tsh[dst]) + v[dst] ** 2 * (y[dst] * jnp.sin(d_ji[dst]) - bsh[dst] / 2)
        delta_q = delta_q_start + seg(q_msg_from, dst) + seg(q_msg_to, src)
        total_loss = total_loss + _GAMMA ** (_K - k) * jnp.mean(delta_p ** 2 + delta_q ** 2)
    return v, theta, total_loss


# speedup vs baseline: 3.2588x; 3.2588x over previous
import functools

import jax
import jax.numpy as jnp
from jax import lax
from jax.experimental import pallas as pl
from jax.experimental.pallas import tpu as pltpu
from jax.experimental.pallas import tpu_sc as plsc

_LATENT = 10
_K = 5
_GAMMA = 0.9
_NW = 32
_CH = 5000


def _blk(p, x):
    x = x @ p['W1'].T + p['b1']
    x = jax.nn.leaky_relu(x, 0.01)
    x = x @ p['W2'].T + p['b2']
    x = jax.nn.leaky_relu(x, 0.01)
    return x @ p['W4'].T + p['b4']


def _sc_multi_gather(tables, idx_arrays, plan, ch=_CH):
    E = idx_arrays[0].shape[0]
    per_w = E // _NW
    n_ch = per_w // ch
    mesh = plsc.VectorSubcoreMesh(core_axis_name="c", subcore_axis_name="s")
    n_tab = len(tables)
    n_idx = len(idx_arrays)

    @functools.partial(
        pl.kernel, mesh=mesh,
        out_type=tuple(jax.ShapeDtypeStruct((E,), tables[t].dtype)
                       for t, _ in plan),
        scratch_types=[
            pltpu.VMEM((ch,), jnp.int32),
            pltpu.VMEM((ch,), jnp.float32),
            pltpu.VMEM((ch,), jnp.int32),
            pltpu.SemaphoreType.DMA,
        ],
    )
    def k(*refs):
        tabs = refs[:n_tab]
        idxs = refs[n_tab:n_tab + n_idx]
        outs = refs[n_tab + n_idx:n_tab + n_idx + len(plan)]
        idx_v, buf_f, buf_i, sem = refs[n_tab + n_idx + len(plan):]
        wid = lax.axis_index("s") * 2 + lax.axis_index("c")
        base0 = wid * per_w

        def body(j, carry):
            base = base0 + j * ch
            sl = pl.ds(base, ch)
            for xi in range(n_idx):
                entries = [(t, p) for p, (t, x) in enumerate(plan) if x == xi]
                if not entries:
                    continue
                pltpu.sync_copy(idxs[xi].at[sl], idx_v)
                for t, p in entries:
                    buf = buf_i if tabs[t].dtype == jnp.int32 else buf_f
                    pltpu.async_copy(tabs[t].at[idx_v], buf, sem).wait()
                    pltpu.sync_copy(buf, outs[p].at[sl])
            return carry

        lax.fori_loop(0, n_ch, body, 0)

    return k(*tables, *idx_arrays)


def kernel(buses, lines, generators, B, L, G, params):
    n = buses.shape[0]
    src = lines[:, L['f_bus']].astype(jnp.int32) - 1
    dst = lines[:, L['t_bus']].astype(jnp.int32) - 1
    gen_idcs = generators[:, G['bus_i']].astype(jnp.int32) - 1
    seg = lambda x, i: jax.ops.segment_sum(x, i, num_segments=n)
    E = src.shape[0]

    sn = src[:n]
    dn = dst[:n]
    r = lines[:, L['r']]; xr = lines[:, L['x']]; tau = lines[:, L['tau']]
    bsh = lines[:, L['b']]; tsh = lines[:, L['theta']]
    y = 1.0 / jnp.sqrt(r ** 2 + xr ** 2)

    const_tabs = [sn, dn, y[:n], tau[:n], tsh[:n], bsh[:n]]
    const_plan = [(0, 0), (1, 0), (0, 1), (1, 1),
                  (2, 0), (3, 0), (4, 0), (5, 0),
                  (2, 1), (3, 1), (4, 1), (5, 1)]
    (ssn, sdn, dsn, ddn,
     y_s, tau_s, tsh_s, bsh_s,
     y_d, tau_d, tsh_d, bsh_d) = _sc_multi_gather(const_tabs, [src, dst],
                                                  const_plan)

    m = jnp.zeros((n, _LATENT), jnp.float32)
    theta = jnp.zeros((n,), jnp.float32)
    total_loss = jnp.float32(0.0)
    v = seg(generators[:, G['vg']], gen_idcs)
    v = jnp.where(v == 0, jnp.ones_like(v), v)
    pg_new = seg(generators[:, G['Pg']], gen_idcs)
    delta_p = pg_new - buses[:, B['Pd']] - buses[:, B['Gs']] * v ** 2
    qg_new = seg(generators[:, G['qg']], gen_idcs)
    delta_q = qg_new - buses[:, B['Qd']] + buses[:, B['Bs']] * v ** 2

    lattr = lines[:, 2:]
    m_dst = jnp.zeros((E, _LATENT), jnp.float32)

    for k in range(_K):
        phi_in = jnp.concatenate([m_dst, lattr], axis=1)
        pv = seg(_blk(params['phi_v'][k], phi_in), dst)
        pt = seg(_blk(params['phi_theta'][k], phi_in), dst)
        pm = seg(_blk(params['phi_m'][k], phi_in), dst)
        nf = jnp.stack([v, theta, delta_p, delta_q], axis=1)
        theta = theta + _blk(params['L_theta'][k], jnp.concatenate([nf, m, pt], 1))[:, 0]
        v = v + _blk(params['L_v'][k], jnp.concatenate([nf, m, pv], 1))[:, 0]
        m = _blk(params['L_m'][k], jnp.concatenate([nf, m, pm], 1))

        last = k == _K - 1
        tabs = [v, theta] + ([] if last else
                             [jnp.ravel(m[:, i]) for i in range(_LATENT)])
        plan = [(0, 0), (1, 0), (0, 1), (1, 1),
                (1, 2), (1, 3), (1, 4), (1, 5)]
        if not last:
            plan += [(2 + i, 1) for i in range(_LATENT)]
        res = _sc_multi_gather(tabs, [src, dst, ssn, sdn, dsn, ddn], plan)
        v_s, th_s, v_d, th_d, th_ssn, th_sdn, th_dsn, th_ddn = res[:8]
        if not last:
            m_dst = jnp.stack(res[8:], axis=1)
        a_s = th_ssn - th_sdn
        aj_d = th_ddn - th_dsn

        d_ij = th_s - th_d
        d_ji = th_d - th_s
        msg = jnp.abs(v_s * v_d * y_s / tau_s * (jnp.sin(d_ij - a_s - tsh_s) + jnp.sin(d_ji - a_s + tsh_s)) + v_s / tau_s ** 2 * y_s * jnp.sin(a_s) + v_d ** 2 * y_s * jnp.sin(a_s))
        p_joule = jnp.sum(seg(msg, dst))
        p_global = jnp.sum(buses[:, B['Pd']]) + jnp.sum(v ** 2 * buses[:, B['Gs']]) + p_joule
        s_set = jnp.sum(generators[:, G['Pg_set']]); s_min = jnp.sum(generators[:, G['Pmin']]); s_max = jnp.sum(generators[:, G['Pmax']])
        lam = jnp.where(p_global < s_set, (p_global - s_min) / (2 * (s_set - s_min)), (p_global - 2 * s_set + s_max) / (2 * (s_max - s_set)))
        pg_k = jnp.where(lam < 0.5, generators[:, G['Pmin']] + 2 * (generators[:, G['Pg_set']] - generators[:, G['Pmin']]) * lam, 2 * generators[:, G['Pg_set']] - generators[:, G['Pmax']] + 2 * (generators[:, G['Pmax']] - generators[:, G['Pg_set']]) * lam)
        qg_start = buses[:, B['Qd']] - buses[:, B['Bs']] * v ** 2
        msg_from = -v_s * v_d * y_s / tau_s * jnp.cos(d_ij - a_s - tsh_s) + (v_s / tau_s) ** 2 * (y_s * jnp.cos(a_s) - bsh_s / 2)
        msg_to = -v_d * v_s * y_d / tau_d * jnp.cos(d_ji - aj_d - tsh_d) + v_d ** 2 * (y_d * jnp.sin(aj_d) - bsh_d / 2)
        aggr_from = seg(msg_from, dst)
        aggr_to = seg(msg_to, src)
        qg_k = qg_start - aggr_from - aggr_to

        delta_p_gens = seg(pg_k, gen_idcs)
        delta_p_start = delta_p_gens - buses[:, B['Pd']] - buses[:, B['Gs']] * v ** 2
        delta_q_start = qg_k - buses[:, B['Qd']] + buses[:, B['Bs']] * v ** 2
        p_msg_from = v_s * v_d * y_s / tau_s * jnp.sin(d_ij - a_s - tsh_s) + (v_s / tau_s) ** 2 * y_s * jnp.sin(a_s)
        p_msg_to = v_d * v_s * y_d / tau_d * jnp.sin(d_ji - aj_d - tsh_d) + v_d ** 2 * y_d * jnp.sin(aj_d)
        delta_p = delta_p_start + seg(p_msg_from, dst) + seg(p_msg_to, src)
        q_msg_from = -v_s * v_d * y_s / tau_s * jnp.cos(d_ij - a_s - tsh_s) + (v_s / tau_s) ** 2 * (y_s * jnp.cos(a_s) - bsh_s / 2)
        q_msg_to = -v_d * v_s * y_d / tau_d * jnp.cos(d_ji - aj_d - tsh_d) + v_d ** 2 * (y_d * jnp.sin(aj_d) - bsh_d / 2)
        delta_q = delta_q_start + seg(q_msg_from, dst) + seg(q_msg_to, src)
        total_loss = total_loss + _GAMMA ** (_K - k) * jnp.mean(delta_p ** 2 + delta_q ** 2)
    return v, theta, total_loss
